# baseline (speedup 1.0000x reference)
import functools

import jax
import jax.numpy as jnp
from jax import lax
from jax.experimental import pallas as pl
from jax.experimental.pallas import tpu as pltpu
from jax.experimental.pallas import tpu_sc as plsc

L = 16
FW_PAD = 128


def _tc_matvec_body(emb_ref, ftpad_ref, w_ref, embw_ref, fieldw_ref):
    w1r = w_ref[0:1, 0:64]
    w2r = w_ref[0:1, 64:128]
    xt = jnp.transpose(emb_ref[...], (1, 0))
    embw_ref[...] = jnp.dot(w1r, xt,
                            preferred_element_type=jnp.float32)[0]

    @pl.when(pl.program_id(0) == 0)
    def _():
        ft = jnp.transpose(ftpad_ref[...], (1, 0))
        fieldw_ref[...] = jnp.dot(w2r, ft,
                                  preferred_element_type=jnp.float32)[0]


def _tc_matvec(emb_table, ft_pad, W):
    V = emb_table.shape[0]
    VB = 40960
    grid = (V + VB - 1) // VB
    return pl.pallas_call(
        _tc_matvec_body,
        grid=(grid,),
        in_specs=[
            pl.BlockSpec((VB, 64), lambda i: (i, 0)),
            pl.BlockSpec((FW_PAD, 64), lambda i: (0, 0)),
            pl.BlockSpec((1, 128), lambda i: (0, 0)),
        ],
        out_specs=[
            pl.BlockSpec((VB,), lambda i: (i,)),
            pl.BlockSpec((FW_PAD,), lambda i: (0,)),
        ],
        out_shape=[
            jax.ShapeDtypeStruct((V,), jnp.float32),
            jax.ShapeDtypeStruct((FW_PAD,), jnp.float32),
        ],
    )(emb_table, ft_pad, W)


def _make_sc_lookup(B, F):
    NC, NS = 2, 16
    NW = NC * NS
    rows_per_w = B // NW
    groups = rows_per_w // L
    assert groups % 2 == 0
    E = L * F
    NG_FULL, REM = divmod(E, 128)
    mesh = plsc.VectorSubcoreMesh(core_axis_name="c", subcore_axis_name="s",
                                  num_cores=NC, num_subcores=NS)

    @functools.partial(
        pl.kernel,
        out_type=jax.ShapeDtypeStruct((B,), jnp.float32),
        mesh=mesh,
        compiler_params=pltpu.CompilerParams(needs_layout_passes=False),
        scratch_types=[
            pltpu.VMEM((E,), jnp.int32),
            pltpu.VMEM((E,), jnp.int32),
            pltpu.VMEM((E,), jnp.int32),
            pltpu.VMEM((E,), jnp.int32),
            pltpu.VMEM((E,), jnp.float32),
            pltpu.VMEM((E,), jnp.float32),
            pltpu.VMEM((E,), jnp.float32),
            pltpu.VMEM((E,), jnp.float32),
            pltpu.VMEM((FW_PAD,), jnp.float32),
            pltpu.VMEM((L,), jnp.float32),
            pltpu.VMEM((rows_per_w,), jnp.float32),
            pltpu.VMEM((25000,), jnp.float32),
            pltpu.VMEM_SHARED((1000000,), jnp.float32),
            pltpu.SemaphoreType.DMA,
            pltpu.SemaphoreType.DMA,
            pltpu.SemaphoreType.DMA,
            pltpu.SemaphoreType.DMA,
        ],
    )
    def sc_lookup(idx_hbm, fld_hbm, val_hbm, embw_hbm, fieldw_hbm, b_hbm,
                  out_hbm, idx0, idx1, fld0, fld1, val0, val1, g0b, g1b,
                  fw_v, b_v, out_v, bounce, embw_sh, semc0, semc1, semg0,
                  semg1):
        wid = lax.axis_index("s") * NC + lax.axis_index("c")
        sid = lax.axis_index("s")
        base = wid * rows_per_w
        @pl.when(sid < 8)
        def _():
            for k in range(5):
                o = sid * 125000 + k * 25000
                pltpu.sync_copy(embw_hbm.at[pl.dslice(o, 25000)], bounce)
                pltpu.sync_copy(bounce, embw_sh.at[pl.dslice(o, 25000)])
        pltpu.sync_copy(fieldw_hbm, fw_v)
        pltpu.sync_copy(b_hbm, b_v)
        plsc.subcore_barrier()
        iota = lax.iota(jnp.int32, L)

        idx_c = (idx0, idx1)
        fld_c = (fld0, fld1)
        val_c = (val0, val1)
        g_c = (g0b, g1b)
        semc = (semc0, semc1)
        semg = (semg0, semg1)

        def fire_chunks(g, p):
            e0 = (base + g * L) * F
            pltpu.async_copy(idx_hbm.at[pl.ds(e0, E)], idx_c[p], semc[p])
            pltpu.async_copy(fld_hbm.at[pl.ds(e0, E)], fld_c[p], semc[p])
            pltpu.async_copy(val_hbm.at[pl.ds(e0, E)], val_c[p], semc[p])

        def wait_chunks(p):
            pltpu.make_async_copy(idx_hbm.at[pl.ds(0, E)], idx_c[p],
                                  semc[p]).wait()
            pltpu.make_async_copy(fld_hbm.at[pl.ds(0, E)], fld_c[p],
                                  semc[p]).wait()
            pltpu.make_async_copy(val_hbm.at[pl.ds(0, E)], val_c[p],
                                  semc[p]).wait()

        def fire_gathers(p):
            for k in range(NG_FULL):
                pltpu.async_copy(embw_sh.at[idx_c[p].at[pl.ds(k * 128, 128)]],
                                 g_c[p].at[pl.ds(k * 128, 128)], semg[p])
            if REM:
                pltpu.async_copy(
                    embw_sh.at[idx_c[p].at[pl.ds(NG_FULL * 128, REM)]],
                    g_c[p].at[pl.ds(NG_FULL * 128, REM)], semg[p])

        def wait_gathers(p):
            for k in range(NG_FULL):
                pltpu.make_async_copy(
                    embw_sh.at[idx_c[p].at[pl.ds(k * 128, 128)]],
                    g_c[p].at[pl.ds(k * 128, 128)], semg[p]).wait()
            if REM:
                pltpu.make_async_copy(
                    embw_sh.at[idx_c[p].at[pl.ds(NG_FULL * 128, REM)]],
                    g_c[p].at[pl.ds(NG_FULL * 128, REM)], semg[p]).wait()

        def compute(g, p):
            acc = b_v[...]
            flat = iota * F
            for f in range(F):
                fi = flat + f
                gv = plsc.load_gather(g_c[p], [fi])
                fldv = plsc.load_gather(fld_c[p], [fi])
                fwv = plsc.load_gather(fw_v, [fldv])
                vv = plsc.load_gather(val_c[p], [fi])
                acc = acc + vv * (gv + fwv)
            out_v[pl.ds(g * L, L)] = acc

        fire_chunks(0, 0)
        wait_chunks(0)
        fire_gathers(0)
        fire_chunks(1, 1)

        def body(i, carry):
            ge = 2 * i
            go = ge + 1
            wait_chunks(1)
            fire_gathers(1)
            wait_gathers(0)
            compute(ge, 0)

            @pl.when(ge + 2 < groups)
            def _():
                fire_chunks(ge + 2, 0)

            wait_gathers(1)
            compute(go, 1)

            @pl.when(ge + 2 < groups)
            def _():
                wait_chunks(0)
                fire_gathers(0)

            @pl.when(ge + 3 < groups)
            def _():
                fire_chunks(ge + 3, 1)

            return carry

        lax.fori_loop(0, groups // 2, body, 0)
        pltpu.sync_copy(out_v, out_hbm.at[pl.ds(base, rows_per_w)])

    return sc_lookup


def kernel(index, field, value, emb_table, field_table, W, b):
    B, F = index.shape
    ft_pad = jnp.zeros((FW_PAD, 64), jnp.float32).at[0:field_table.shape[0]].set(
        field_table)
    embw, fieldw = _tc_matvec(emb_table, ft_pad, W.reshape(1, 128))
    b16 = jnp.broadcast_to(b, (L,))
    out = _make_sc_lookup(B, F)(index.reshape(-1), field.reshape(-1),
                                value.reshape(-1), embw, fieldw, b16)
    return out[:, None]


# speedup vs baseline: 1.0009x; 1.0009x over previous
import functools

import jax
import jax.numpy as jnp
from jax import lax
from jax.experimental import pallas as pl
from jax.experimental.pallas import tpu as pltpu
from jax.experimental.pallas import tpu_sc as plsc

L = 16
FW_PAD = 128


def _tc_matvec_body(emb_ref, ftpad_ref, w_ref, embw_ref, fieldw_ref):
    w1r = w_ref[0:1, 0:64]
    w2r = w_ref[0:1, 64:128]
    xt = jnp.transpose(emb_ref[...], (1, 0))
    embw_ref[...] = jnp.dot(w1r, xt,
                            preferred_element_type=jnp.float32)[0]

    @pl.when(pl.program_id(0) == 0)
    def _():
        ft = jnp.transpose(ftpad_ref[...], (1, 0))
        fieldw_ref[...] = jnp.dot(w2r, ft,
                                  preferred_element_type=jnp.float32)[0]


def _tc_matvec(emb_table, ft_pad, W):
    V = emb_table.shape[0]
    VB = 32768
    grid = (V + VB - 1) // VB
    return pl.pallas_call(
        _tc_matvec_body,
        grid=(grid,),
        in_specs=[
            pl.BlockSpec((VB, 64), lambda i: (i, 0)),
            pl.BlockSpec((FW_PAD, 64), lambda i: (0, 0)),
            pl.BlockSpec((1, 128), lambda i: (0, 0)),
        ],
        out_specs=[
            pl.BlockSpec((VB,), lambda i: (i,)),
            pl.BlockSpec((FW_PAD,), lambda i: (0,)),
        ],
        out_shape=[
            jax.ShapeDtypeStruct((V,), jnp.float32),
            jax.ShapeDtypeStruct((FW_PAD,), jnp.float32),
        ],
    )(emb_table, ft_pad, W)


def _make_sc_lookup(B, F):
    NC, NS = 2, 16
    NW = NC * NS
    rows_per_w = B // NW
    groups = rows_per_w // L
    assert groups % 2 == 0
    E = L * F
    NG_FULL, REM = divmod(E, 128)
    mesh = plsc.VectorSubcoreMesh(core_axis_name="c", subcore_axis_name="s",
                                  num_cores=NC, num_subcores=NS)

    @functools.partial(
        pl.kernel,
        out_type=jax.ShapeDtypeStruct((B,), jnp.float32),
        mesh=mesh,
        compiler_params=pltpu.CompilerParams(needs_layout_passes=False),
        scratch_types=[
            pltpu.VMEM((E,), jnp.int32),
            pltpu.VMEM((E,), jnp.int32),
            pltpu.VMEM((E,), jnp.int32),
            pltpu.VMEM((E,), jnp.int32),
            pltpu.VMEM((E,), jnp.float32),
            pltpu.VMEM((E,), jnp.float32),
            pltpu.VMEM((E,), jnp.float32),
            pltpu.VMEM((E,), jnp.float32),
            pltpu.VMEM((FW_PAD,), jnp.float32),
            pltpu.VMEM((L,), jnp.float32),
            pltpu.VMEM((rows_per_w,), jnp.float32),
            pltpu.VMEM((25000,), jnp.float32),
            pltpu.VMEM_SHARED((1000000,), jnp.float32),
            pltpu.SemaphoreType.DMA,
            pltpu.SemaphoreType.DMA,
            pltpu.SemaphoreType.DMA,
            pltpu.SemaphoreType.DMA,
        ],
    )
    def sc_lookup(idx_hbm, fld_hbm, val_hbm, embw_hbm, fieldw_hbm, b_hbm,
                  out_hbm, idx0, idx1, fld0, fld1, val0, val1, g0b, g1b,
                  fw_v, b_v, out_v, bounce, embw_sh, semc0, semc1, semg0,
                  semg1):
        wid = lax.axis_index("s") * NC + lax.axis_index("c")
        sid = lax.axis_index("s")
        base = wid * rows_per_w
        @pl.when(sid < 8)
        def _():
            for k in range(5):
                o = sid * 125000 + k * 25000
                pltpu.sync_copy(embw_hbm.at[pl.dslice(o, 25000)], bounce)
                pltpu.sync_copy(bounce, embw_sh.at[pl.dslice(o, 25000)])
        pltpu.sync_copy(fieldw_hbm, fw_v)
        pltpu.sync_copy(b_hbm, b_v)
        plsc.subcore_barrier()
        iota = lax.iota(jnp.int32, L)

        idx_c = (idx0, idx1)
        fld_c = (fld0, fld1)
        val_c = (val0, val1)
        g_c = (g0b, g1b)
        semc = (semc0, semc1)
        semg = (semg0, semg1)

        def fire_chunks(g, p):
            e0 = (base + g * L) * F
            pltpu.async_copy(idx_hbm.at[pl.ds(e0, E)], idx_c[p], semc[p])
            pltpu.async_copy(fld_hbm.at[pl.ds(e0, E)], fld_c[p], semc[p])
            pltpu.async_copy(val_hbm.at[pl.ds(e0, E)], val_c[p], semc[p])

        def wait_chunks(p):
            pltpu.make_async_copy(idx_hbm.at[pl.ds(0, E)], idx_c[p],
                                  semc[p]).wait()
            pltpu.make_async_copy(fld_hbm.at[pl.ds(0, E)], fld_c[p],
                                  semc[p]).wait()
            pltpu.make_async_copy(val_hbm.at[pl.ds(0, E)], val_c[p],
                                  semc[p]).wait()

        def fire_gathers(p):
            for k in range(NG_FULL):
                pltpu.async_copy(embw_sh.at[idx_c[p].at[pl.ds(k * 128, 128)]],
                                 g_c[p].at[pl.ds(k * 128, 128)], semg[p])
            if REM:
                pltpu.async_copy(
                    embw_sh.at[idx_c[p].at[pl.ds(NG_FULL * 128, REM)]],
                    g_c[p].at[pl.ds(NG_FULL * 128, REM)], semg[p])

        def wait_gathers(p):
            for k in range(NG_FULL):
                pltpu.make_async_copy(
                    embw_sh.at[idx_c[p].at[pl.ds(k * 128, 128)]],
                    g_c[p].at[pl.ds(k * 128, 128)], semg[p]).wait()
            if REM:
                pltpu.make_async_copy(
                    embw_sh.at[idx_c[p].at[pl.ds(NG_FULL * 128, REM)]],
                    g_c[p].at[pl.ds(NG_FULL * 128, REM)], semg[p]).wait()

        def compute(g, p):
            acc = b_v[...]
            flat = iota * F
            for f in range(F):
                fi = flat + f
                gv = plsc.load_gather(g_c[p], [fi])
                fldv = plsc.load_gather(fld_c[p], [fi])
                fwv = plsc.load_gather(fw_v, [fldv])
                vv = plsc.load_gather(val_c[p], [fi])
                acc = acc + vv * (gv + fwv)
            out_v[pl.ds(g * L, L)] = acc

        fire_chunks(0, 0)
        wait_chunks(0)
        fire_gathers(0)
        fire_chunks(1, 1)

        def body(i, carry):
            ge = 2 * i
            go = ge + 1
            wait_chunks(1)
            fire_gathers(1)
            wait_gathers(0)
            compute(ge, 0)

            @pl.when(ge + 2 < groups)
            def _():
                fire_chunks(ge + 2, 0)

            wait_gathers(1)
            compute(go, 1)

            @pl.when(ge + 2 < groups)
            def _():
                wait_chunks(0)
                fire_gathers(0)

            @pl.when(ge + 3 < groups)
            def _():
                fire_chunks(ge + 3, 1)

            return carry

        lax.fori_loop(0, groups // 2, body, 0)
        pltpu.sync_copy(out_v, out_hbm.at[pl.ds(base, rows_per_w)])

    return sc_lookup


def kernel(index, field, value, emb_table, field_table, W, b):
    B, F = index.shape
    ft_pad = jnp.zeros((FW_PAD, 64), jnp.float32).at[0:field_table.shape[0]].set(
        field_table)
    embw, fieldw = _tc_matvec(emb_table, ft_pad, W.reshape(1, 128))
    b16 = jnp.broadcast_to(b, (L,))
    out = _make_sc_lookup(B, F)(index.reshape(-1), field.reshape(-1),
                                value.reshape(-1), embw, fieldw, b16)
    return out[:, None]


# speedup vs baseline: 1.0030x; 1.0021x over previous
import functools

import jax
import jax.numpy as jnp
from jax import lax
from jax.experimental import pallas as pl
from jax.experimental.pallas import tpu as pltpu
from jax.experimental.pallas import tpu_sc as plsc

L = 16
FW_PAD = 128


def _tc_matvec_body(emb_ref, ftpad_ref, w_ref, embw_ref, fieldw_ref):
    w1r = w_ref[0:1, 0:64]
    w2r = w_ref[0:1, 64:128]
    xt = jnp.transpose(emb_ref[...], (1, 0))
    embw_ref[...] = jnp.dot(w1r, xt,
                            preferred_element_type=jnp.float32)[0]

    @pl.when(pl.program_id(0) == 0)
    def _():
        ft = jnp.transpose(ftpad_ref[...], (1, 0))
        fieldw_ref[...] = jnp.dot(w2r, ft,
                                  preferred_element_type=jnp.float32)[0]


def _tc_matvec(emb_table, ft_pad, W):
    V = emb_table.shape[0]
    VB = 32768
    grid = (V + VB - 1) // VB
    return pl.pallas_call(
        _tc_matvec_body,
        grid=(grid,),
        in_specs=[
            pl.BlockSpec((VB, 64), lambda i: (i, 0)),
            pl.BlockSpec((FW_PAD, 64), lambda i: (0, 0)),
            pl.BlockSpec((1, 128), lambda i: (0, 0)),
        ],
        out_specs=[
            pl.BlockSpec((VB,), lambda i: (i,)),
            pl.BlockSpec((FW_PAD,), lambda i: (0,)),
        ],
        out_shape=[
            jax.ShapeDtypeStruct((V,), jnp.float32),
            jax.ShapeDtypeStruct((FW_PAD,), jnp.float32),
        ],
    )(emb_table, ft_pad, W)


def _make_sc_lookup(B, F, V):
    NC, NS = 2, 16
    NW = NC * NS
    rows_per_w = B // NW
    groups = rows_per_w // L
    assert groups % 2 == 0
    E = L * F
    STAGERS = 8
    SEG = V // STAGERS
    NSEG = 5
    assert SEG % NSEG == 0 and (SEG // NSEG) % 8 == 0 and SEG * STAGERS == V
    NG_FULL, REM = divmod(E, 128)
    mesh = plsc.VectorSubcoreMesh(core_axis_name="c", subcore_axis_name="s",
                                  num_cores=NC, num_subcores=NS)

    @functools.partial(
        pl.kernel,
        out_type=jax.ShapeDtypeStruct((B,), jnp.float32),
        mesh=mesh,
        compiler_params=pltpu.CompilerParams(needs_layout_passes=False),
        scratch_types=[
            pltpu.VMEM((E,), jnp.int32),
            pltpu.VMEM((E,), jnp.int32),
            pltpu.VMEM((E,), jnp.int32),
            pltpu.VMEM((E,), jnp.int32),
            pltpu.VMEM((E,), jnp.float32),
            pltpu.VMEM((E,), jnp.float32),
            pltpu.VMEM((E,), jnp.float32),
            pltpu.VMEM((E,), jnp.float32),
            pltpu.VMEM((FW_PAD,), jnp.float32),
            pltpu.VMEM((L,), jnp.float32),
            pltpu.VMEM((rows_per_w,), jnp.float32),
            pltpu.VMEM((SEG // NSEG,), jnp.float32),
            pltpu.VMEM_SHARED((V,), jnp.float32),
            pltpu.SemaphoreType.DMA,
            pltpu.SemaphoreType.DMA,
            pltpu.SemaphoreType.DMA,
            pltpu.SemaphoreType.DMA,
        ],
    )
    def sc_lookup(idx_hbm, fld_hbm, val_hbm, embw_hbm, fieldw_hbm, b_hbm,
                  out_hbm, idx0, idx1, fld0, fld1, val0, val1, g0b, g1b,
                  fw_v, b_v, out_v, bounce, embw_sh, semc0, semc1, semg0,
                  semg1):
        wid = lax.axis_index("s") * NC + lax.axis_index("c")
        sid = lax.axis_index("s")
        base = wid * rows_per_w
        @pl.when(sid < STAGERS)
        def _():
            for k in range(NSEG):
                o = sid * SEG + k * (SEG // NSEG)
                pltpu.sync_copy(embw_hbm.at[pl.dslice(o, SEG // NSEG)], bounce)
                pltpu.sync_copy(bounce, embw_sh.at[pl.dslice(o, SEG // NSEG)])
        pltpu.sync_copy(fieldw_hbm, fw_v)
        pltpu.sync_copy(b_hbm, b_v)
        plsc.subcore_barrier()
        iota = lax.iota(jnp.int32, L)

        idx_c = (idx0, idx1)
        fld_c = (fld0, fld1)
        val_c = (val0, val1)
        g_c = (g0b, g1b)
        semc = (semc0, semc1)
        semg = (semg0, semg1)

        def fire_chunks(g, p):
            e0 = (base + g * L) * F
            pltpu.async_copy(idx_hbm.at[pl.ds(e0, E)], idx_c[p], semc[p])
            pltpu.async_copy(fld_hbm.at[pl.ds(e0, E)], fld_c[p], semc[p])
            pltpu.async_copy(val_hbm.at[pl.ds(e0, E)], val_c[p], semc[p])

        def wait_chunks(p):
            pltpu.make_async_copy(idx_hbm.at[pl.ds(0, E)], idx_c[p],
                                  semc[p]).wait()
            pltpu.make_async_copy(fld_hbm.at[pl.ds(0, E)], fld_c[p],
                                  semc[p]).wait()
            pltpu.make_async_copy(val_hbm.at[pl.ds(0, E)], val_c[p],
                                  semc[p]).wait()

        def fire_gathers(p):
            for k in range(NG_FULL):
                pltpu.async_copy(embw_sh.at[idx_c[p].at[pl.ds(k * 128, 128)]],
                                 g_c[p].at[pl.ds(k * 128, 128)], semg[p])
            if REM:
                pltpu.async_copy(
                    embw_sh.at[idx_c[p].at[pl.ds(NG_FULL * 128, REM)]],
                    g_c[p].at[pl.ds(NG_FULL * 128, REM)], semg[p])

        def wait_gathers(p):
            for k in range(NG_FULL):
                pltpu.make_async_copy(
                    embw_sh.at[idx_c[p].at[pl.ds(k * 128, 128)]],
                    g_c[p].at[pl.ds(k * 128, 128)], semg[p]).wait()
            if REM:
                pltpu.make_async_copy(
                    embw_sh.at[idx_c[p].at[pl.ds(NG_FULL * 128, REM)]],
                    g_c[p].at[pl.ds(NG_FULL * 128, REM)], semg[p]).wait()

        def compute(g, p):
            acc = b_v[...]
            flat = iota * F
            for f in range(F):
                fi = flat + f
                gv = plsc.load_gather(g_c[p], [fi])
                fldv = plsc.load_gather(fld_c[p], [fi])
                fwv = plsc.load_gather(fw_v, [fldv])
                vv = plsc.load_gather(val_c[p], [fi])
                acc = acc + vv * (gv + fwv)
            out_v[pl.ds(g * L, L)] = acc

        fire_chunks(0, 0)
        wait_chunks(0)
        fire_gathers(0)
        fire_chunks(1, 1)

        def body(i, carry):
            ge = 2 * i
            go = ge + 1
            wait_chunks(1)
            fire_gathers(1)
            wait_gathers(0)
            compute(ge, 0)

            @pl.when(ge + 2 < groups)
            def _():
                fire_chunks(ge + 2, 0)

            wait_gathers(1)
            compute(go, 1)

            @pl.when(ge + 2 < groups)
            def _():
                wait_chunks(0)
                fire_gathers(0)

            @pl.when(ge + 3 < groups)
            def _():
                fire_chunks(ge + 3, 1)

            return carry

        lax.fori_loop(0, groups // 2, body, 0)
        pltpu.sync_copy(out_v, out_hbm.at[pl.ds(base, rows_per_w)])

    return sc_lookup


def kernel(index, field, value, emb_table, field_table, W, b):
    B, F = index.shape
    ft_pad = jnp.zeros((FW_PAD, 64), jnp.float32).at[0:field_table.shape[0]].set(
        field_table)
    embw, fieldw = _tc_matvec(emb_table, ft_pad, W.reshape(1, 128))
    b16 = jnp.broadcast_to(b, (L,))
    out = _make_sc_lookup(B, F, emb_table.shape[0])(
        index.reshape(-1), field.reshape(-1), value.reshape(-1),
        embw, fieldw, b16)
    return out[:, None]
